# Initial kernel scaffold; baseline (speedup 1.0000x reference)
import jax
import jax.numpy as jnp
from jax.experimental import pallas as pl


def kernel(x, edge_index, pan_weight, W1, b1, W2, b2):
    raise NotImplementedError("write your pallas kernel here")


# speedup vs baseline: 154.3987x; 154.3987x over previous
import functools

import jax
import jax.numpy as jnp
from jax import lax
from jax.experimental import pallas as pl
from jax.experimental.pallas import tpu as pltpu
from jax.experimental.pallas import tpu_sc as plsc

N = 10000
E = 320000
C = 128
NC = 2
NS = 16
NW = NC * NS
CH = 128
CPW = 79
PW = CPW * CH
EP = NW * PW
PAD = EP - E
TBL = N * N + PAD
NACC = 16 * 632
NDEG = 10240

def _i32(v):
    return jnp.int32(v)


def _worker(cid, sid):
    return sid * _i32(NC) + cid


def _chunk_loop(body):
    def outer(j, _):
        def inner(k, _):
            body(j, k, pl.ds(k * _i32(16), 16))
            return _i32(0)
        return lax.fori_loop(_i32(0), _i32(8), inner, _i32(0))
    lax.fori_loop(_i32(0), _i32(CPW), outer, _i32(0))


def _compute_keys(rbuf, cbuf, kbuf):
    def body(j, k, sl):
        kbuf[j, sl] = rbuf[j, sl] * _i32(N) + cbuf[j, sl]
    _chunk_loop(body)
    return lax.iota(jnp.int32, 16)


def _k1_body(rows_hbm, cols_hbm, tbl_hbm, rbuf, cbuf, kbuf, ibuf, sem):
    cid = lax.axis_index("c")
    sid = lax.axis_index("s")
    wid = _worker(cid, sid)
    pltpu.sync_copy(rows_hbm.at[wid], rbuf)
    pltpu.sync_copy(cols_hbm.at[wid], cbuf)
    iota = _compute_keys(rbuf, cbuf, kbuf)
    base = wid * _i32(PW)

    def fill_ids(j, k, sl):
        ibuf[j, sl] = base + j * _i32(128) + k * _i32(16) + iota

    _chunk_loop(fill_ids)

    def scat(j, _):
        pltpu.async_copy(ibuf.at[j], tbl_hbm.at[kbuf.at[j]], sem).wait()
        return _i32(0)

    lax.fori_loop(_i32(0), _i32(CPW), scat, _i32(0))


def _k2_body(rows_hbm, cols_hbm, tbl_hbm, degp_hbm,
             rbuf, cbuf, kbuf, vbuf, xbuf, gbuf, zbuf, degS, sem):
    cid = lax.axis_index("c")
    sid = lax.axis_index("s")
    wid = _worker(cid, sid)
    pltpu.sync_copy(rows_hbm.at[wid], rbuf)
    pltpu.sync_copy(cols_hbm.at[wid], cbuf)
    iota = _compute_keys(rbuf, cbuf, kbuf)
    base = wid * _i32(PW)

    def zero(i, _):
        zbuf[pl.ds(i * _i32(16), 16)] = jnp.zeros((16,), jnp.float32)
        return _i32(0)

    lax.fori_loop(_i32(0), _i32(40), zero, _i32(0))
    pltpu.sync_copy(zbuf, degS.at[pl.ds(sid * _i32(640), 640)])
    plsc.subcore_barrier()

    ones = jnp.full((16,), 1.0, jnp.float32)
    zeros = jnp.zeros((16,), jnp.float32)
    zi = jnp.zeros((16,), jnp.int32)

    def proc(j, _):
        pltpu.async_copy(tbl_hbm.at[kbuf.at[j]], gbuf, sem).wait()

        def inner(k, _):
            sl = pl.ds(k * _i32(16), 16)
            r = rbuf[j, sl]
            c = cbuf[j, sl]
            g = gbuf[sl]
            myid = base + j * _i32(128) + k * _i32(16) + iota
            valid = r < _i32(N)
            flag = (g == myid) & (r != c) & valid
            vbuf[j, sl] = jnp.where(flag, ones, zeros)
            xbuf[j, sl] = jnp.where(valid, r, zi)
            return _i32(0)

        lax.fori_loop(_i32(0), _i32(8), inner, _i32(0))
        pltpu.sync_copy(vbuf.at[j], degS.at[xbuf.at[j]], add=True)
        return _i32(0)

    lax.fori_loop(_i32(0), _i32(CPW), proc, _i32(0))
    plsc.subcore_barrier()
    pltpu.sync_copy(degS.at[pl.ds(sid * _i32(640), 640)],
                    degp_hbm.at[cid, pl.ds(sid * _i32(640), 640)])


def _k4_body(rows_hbm, cols_hbm, xs_hbm, acc_hbm,
             rbuf, cbuf, rowsbuf, accS, sem):
    cid = lax.axis_index("c")
    sid = lax.axis_index("s")
    wid = _worker(cid, sid)
    pltpu.sync_copy(rows_hbm.at[wid], rbuf)
    pltpu.sync_copy(cols_hbm.at[wid], cbuf)
    iota = lax.iota(jnp.int32, 16)

    def mk_sidx(j, k, sl):
        r = rbuf[j, sl]
        rbuf[j, sl] = jnp.where(r < _i32(N), r, _i32(N) + iota)

    _chunk_loop(mk_sidx)

    z16 = jnp.zeros((16,), jnp.float32)

    def zero(i, _):
        def zin(k, _):
            rowsbuf[i, pl.ds(k * _i32(16), 16)] = z16
            return _i32(0)
        return lax.fori_loop(_i32(0), _i32(8), zin, _i32(0))

    lax.fori_loop(_i32(0), _i32(128), zero, _i32(0))
    base_row = sid * _i32(632)
    for t in range(4):
        pltpu.sync_copy(rowsbuf, accS.at[pl.ds(base_row + _i32(t * 128), 128)])
    pltpu.sync_copy(rowsbuf.at[pl.ds(0, 120)],
                    accS.at[pl.ds(base_row + _i32(512), 120)])
    plsc.subcore_barrier()

    def proc(j, _):
        pltpu.async_copy(xs_hbm.at[cbuf.at[j]], rowsbuf, sem).wait()
        pltpu.sync_copy(rowsbuf, accS.at[rbuf.at[j]], add=True)
        return _i32(0)

    lax.fori_loop(_i32(0), _i32(CPW), proc, _i32(0))
    plsc.subcore_barrier()
    for t in range(4):
        pltpu.sync_copy(accS.at[pl.ds(base_row + _i32(t * 128), 128)],
                        acc_hbm.at[cid, pl.ds(base_row + _i32(t * 128), 128)])
    pltpu.sync_copy(accS.at[pl.ds(base_row + _i32(512), 120)],
                    acc_hbm.at[cid, pl.ds(base_row + _i32(512), 120)])


@functools.lru_cache(maxsize=1)
def _sc_kernels():
    mesh = plsc.VectorSubcoreMesh(core_axis_name="c", subcore_axis_name="s")
    k1 = pl.kernel(
        _k1_body,
        out_type=jax.ShapeDtypeStruct((TBL,), jnp.int32),
        mesh=mesh,
        scratch_types=[
        pltpu.VMEM((CPW, CH), jnp.int32),
        pltpu.VMEM((CPW, CH), jnp.int32),
        pltpu.VMEM((CPW, CH), jnp.int32),
            pltpu.VMEM((CPW, CH), jnp.int32),
            pltpu.SemaphoreType.DMA,
        ],
    )
    k2 = pl.kernel(
        _k2_body,
        out_type=jax.ShapeDtypeStruct((NC, NDEG), jnp.float32),
        mesh=mesh,
        scratch_types=[
        pltpu.VMEM((CPW, CH), jnp.int32),
        pltpu.VMEM((CPW, CH), jnp.int32),
        pltpu.VMEM((CPW, CH), jnp.int32),
        pltpu.VMEM((CPW, CH), jnp.float32),
        pltpu.VMEM((CPW, CH), jnp.int32),
        pltpu.VMEM((CH,), jnp.int32),
            pltpu.VMEM((640,), jnp.float32),
            pltpu.VMEM_SHARED((NDEG,), jnp.float32),
            pltpu.SemaphoreType.DMA,
        ],
    )
    k4 = pl.kernel(
        _k4_body,
        out_type=jax.ShapeDtypeStruct((NC, NACC, C), jnp.float32),
        mesh=mesh,
        scratch_types=[
            pltpu.VMEM((CPW, CH), jnp.int32),
            pltpu.VMEM((CPW, CH), jnp.int32),
            pltpu.VMEM((CH, C), jnp.float32),
            pltpu.VMEM_SHARED((NACC, C), jnp.float32),
            pltpu.SemaphoreType.DMA,
        ],
    )
    return k1, k2, k4


def _z():
    return jnp.int32(0)


_BLK = 1000


def _k3_body(degp_ref, x_ref, xs_ref, disb_ref):
    dp = degp_ref[...]
    deg = 1.0 + dp[:, 0:1] + dp[:, 1:2]
    dis = lax.rsqrt(deg)
    disb_ref[...] = jnp.broadcast_to(dis, (_BLK, C))
    xs_ref[...] = dis * x_ref[...]


def _k5_body(acc_ref, disb_ref, xs_ref, pan_ref, w1_ref, b1_ref,
             w2_ref, b2_ref, out_ref):
    a = acc_ref[...]
    s = a[0] + a[1]
    w0 = pan_ref[0, 0]
    w01 = pan_ref[0, 1]
    y = disb_ref[...] * (w01 * s + w0 * xs_ref[...])
    dn = (((1,), (1,)), ((), ()))
    z = lax.dot_general(y, w1_ref[...], dn,
                        preferred_element_type=jnp.float32) + b1_ref[...]
    h = jnp.where(z > 0, z, jnp.exp(z) - 1.0)
    out_ref[...] = lax.dot_general(h, w2_ref[...], dn,
                                   preferred_element_type=jnp.float32) + b2_ref[...]


_k3 = pl.pallas_call(
    _k3_body,
    grid=(N // _BLK,),
    in_specs=[
        pl.BlockSpec((_BLK, 2), lambda j: (j, _z())),
        pl.BlockSpec((_BLK, C), lambda j: (j, _z())),
    ],
    out_specs=[
        pl.BlockSpec((_BLK, C), lambda j: (j, _z())),
        pl.BlockSpec((_BLK, C), lambda j: (j, _z())),
    ],
    out_shape=[
        jax.ShapeDtypeStruct((N, C), jnp.float32),
        jax.ShapeDtypeStruct((N, C), jnp.float32),
    ],
)

_k5 = pl.pallas_call(
    _k5_body,
    grid=(N // _BLK,),
    in_specs=[
        pl.BlockSpec((NC, _BLK, C), lambda j: (_z(), j, _z())),
        pl.BlockSpec((_BLK, C), lambda j: (j, _z())),
        pl.BlockSpec((_BLK, C), lambda j: (j, _z())),
        pl.BlockSpec((1, 2), lambda j: (_z(), _z())),
        pl.BlockSpec((C, C), lambda j: (_z(), _z())),
        pl.BlockSpec((1, C), lambda j: (_z(), _z())),
        pl.BlockSpec((C, C), lambda j: (_z(), _z())),
        pl.BlockSpec((1, C), lambda j: (_z(), _z())),
    ],
    out_specs=pl.BlockSpec((_BLK, C), lambda j: (j, _z())),
    out_shape=jax.ShapeDtypeStruct((N, C), jnp.float32),
)


def kernel(x, edge_index, pan_weight, W1, b1, W2, b2):
    ei = edge_index.astype(jnp.int32)
    rows = jnp.concatenate(
        [ei[1], jnp.full((PAD,), N, jnp.int32)]).reshape(NW, CPW, CH)
    cols = jnp.concatenate(
        [ei[0], jnp.arange(PAD, dtype=jnp.int32)]).reshape(NW, CPW, CH)
    x = x.astype(jnp.float32)
    k1, k2, k4 = _sc_kernels()
    tbl = k1(rows, cols)
    degp = k2(rows, cols, tbl)
    xs, disb = _k3(degp.T, x)
    acc = k4(rows, cols, xs)
    pw = pan_weight.astype(jnp.float32)
    pan2 = jnp.stack([pw[0], pw[0] * pw[1]]).reshape(1, 2)
    out = _k5(acc, disb, xs, pan2,
              W1.astype(jnp.float32), b1.astype(jnp.float32).reshape(1, C),
              W2.astype(jnp.float32), b2.astype(jnp.float32).reshape(1, C))
    return out.astype(jnp.float64)


# speedup vs baseline: 189.5943x; 1.2280x over previous
import functools

import jax
import jax.numpy as jnp
from jax import lax
from jax.experimental import pallas as pl
from jax.experimental.pallas import tpu as pltpu
from jax.experimental.pallas import tpu_sc as plsc

N = 10000
E = 320000
C = 128
NC = 2
NS = 16
NW = NC * NS
CH = 96
CPW = 105
PW = CPW * CH
EP = NW * PW
PAD = EP - E
TBL = N * N + PAD
SPT = 632
NACC = NS * SPT
NDEG = 10240
CH4 = 96
CPW4 = PW // CH4


def _i32(v):
    return jnp.int32(v)


def _worker(cid, sid):
    return sid * _i32(NC) + cid


def _chunk_loop(body):
    def outer(j, _):
        def inner(k, _):
            body(j, k, pl.ds(k * _i32(16), 16))
            return _i32(0)
        return lax.fori_loop(_i32(0), _i32(CH // 16), inner, _i32(0))
    lax.fori_loop(_i32(0), _i32(CPW), outer, _i32(0))


def _compute_keys(rbuf, cbuf, kbuf):
    def body(j, k, sl):
        kbuf[j, sl] = rbuf[j, sl] * _i32(N) + cbuf[j, sl]
    _chunk_loop(body)
    return lax.iota(jnp.int32, 16)


def _k1_body(rows_hbm, cols_hbm, tbl_hbm, rbuf, cbuf, kbuf, ibuf, sem):
    cid = lax.axis_index("c")
    sid = lax.axis_index("s")
    wid = _worker(cid, sid)
    pltpu.sync_copy(rows_hbm.at[wid], rbuf)
    pltpu.sync_copy(cols_hbm.at[wid], cbuf)
    iota = lax.iota(jnp.int32, 16)
    base = wid * _i32(PW)

    def body(i, _):
        sl = pl.ds(i * _i32(16), 16)
        kbuf[sl] = rbuf[sl] * _i32(N) + cbuf[sl]
        ibuf[sl] = base + i * _i32(16) + iota
        return _i32(0)

    lax.fori_loop(_i32(0), _i32(PW // 16), body, _i32(0))
    pltpu.async_copy(ibuf, tbl_hbm.at[kbuf], sem).wait()


def _k2_body(rows_hbm, cols_hbm, tbl_hbm, degp_hbm,
             rbuf, cbuf, kbuf, vbuf, xbuf, gbuf, zbuf, degS, sem):
    cid = lax.axis_index("c")
    sid = lax.axis_index("s")
    wid = _worker(cid, sid)
    pltpu.sync_copy(rows_hbm.at[wid], rbuf)
    pltpu.sync_copy(cols_hbm.at[wid], cbuf)
    iota = lax.iota(jnp.int32, 16)
    base = wid * _i32(PW)

    def keys(i, _):
        sl = pl.ds(i * _i32(16), 16)
        kbuf[sl] = rbuf[sl] * _i32(N) + cbuf[sl]
        return _i32(0)

    lax.fori_loop(_i32(0), _i32(PW // 16), keys, _i32(0))
    gcopy = pltpu.async_copy(tbl_hbm.at[kbuf], gbuf, sem)

    def zero(i, _):
        zbuf[pl.ds(i * _i32(16), 16)] = jnp.zeros((16,), jnp.float32)
        return _i32(0)

    lax.fori_loop(_i32(0), _i32(40), zero, _i32(0))
    pltpu.sync_copy(zbuf, degS.at[pl.ds(sid * _i32(640), 640)])
    gcopy.wait()
    plsc.subcore_barrier()

    ones = jnp.full((16,), 1.0, jnp.float32)
    zeros = jnp.zeros((16,), jnp.float32)
    zi = jnp.zeros((16,), jnp.int32)

    def flags(i, _):
        sl = pl.ds(i * _i32(16), 16)
        r = rbuf[sl]
        c = cbuf[sl]
        g = gbuf[sl]
        myid = base + i * _i32(16) + iota
        valid = r < _i32(N)
        flag = (g == myid) & (r != c) & valid
        vbuf[sl] = jnp.where(flag, ones, zeros)
        xbuf[sl] = jnp.where(valid, r, zi)
        return _i32(0)

    lax.fori_loop(_i32(0), _i32(PW // 16), flags, _i32(0))
    pltpu.sync_copy(vbuf, degS.at[xbuf], add=True)
    plsc.subcore_barrier()
    pltpu.sync_copy(degS.at[pl.ds(sid * _i32(640), 640)],
                    degp_hbm.at[cid, pl.ds(sid * _i32(640), 640)])


def _k4_body(rows_hbm, cols_hbm, xs_hbm, acc_hbm,
             cbuf, ring, rowsbuf, accS, sem, sem2):
    cid = lax.axis_index("c")
    sid = lax.axis_index("s")
    wid = _worker(cid, sid)
    pltpu.sync_copy(cols_hbm.at[wid], cbuf)
    iota = lax.iota(jnp.int32, 16)

    def xform(b):
        def inner(k, _):
            sl = pl.ds(k * _i32(16), 16)
            r = ring[b, sl]
            ring[b, sl] = jnp.where(r < _i32(N), r, _i32(N) + iota)
            return _i32(0)
        lax.fori_loop(_i32(0), _i32(CH4 // 16), inner, _i32(0))

    z16 = jnp.zeros((16,), jnp.float32)

    def zero(i, _):
        def zin(k, _):
            rowsbuf[_i32(0), i, pl.ds(k * _i32(16), 16)] = z16
            return _i32(0)
        return lax.fori_loop(_i32(0), _i32(8), zin, _i32(0))

    lax.fori_loop(_i32(0), _i32(CH4), zero, _i32(0))
    base_row = sid * _i32(SPT)
    for t in range(6):
        pltpu.sync_copy(rowsbuf.at[_i32(0)],
                        accS.at[pl.ds(base_row + _i32(t * CH4), CH4)])
    pltpu.sync_copy(rowsbuf.at[_i32(0), pl.ds(0, SPT - 6 * CH4)],
                    accS.at[pl.ds(base_row + _i32(6 * CH4), SPT - 6 * CH4)])
    plsc.subcore_barrier()

    pltpu.sync_copy(rows_hbm.at[wid, _i32(0)], ring.at[_i32(0)])
    xform(_i32(0))
    pltpu.async_copy(xs_hbm.at[cbuf.at[_i32(0)]], rowsbuf.at[_i32(0)], sem)

    def proc(j, _):
        b = j & _i32(1)
        nxt = j + _i32(1)
        bn = nxt & _i32(1)

        @pl.when(nxt < _i32(CPW4))
        def _():
            pltpu.async_copy(xs_hbm.at[cbuf.at[nxt]], rowsbuf.at[bn], sem)
            pltpu.async_copy(rows_hbm.at[wid, nxt], ring.at[bn], sem2)

        pltpu.make_async_copy(xs_hbm.at[cbuf.at[j]],
                              rowsbuf.at[b], sem).wait()
        pltpu.sync_copy(rowsbuf.at[b], accS.at[ring.at[b]], add=True)

        @pl.when(nxt < _i32(CPW4))
        def _():
            pltpu.make_async_copy(rows_hbm.at[wid, nxt],
                                  ring.at[bn], sem2).wait()
            xform(bn)

        return _i32(0)

    lax.fori_loop(_i32(0), _i32(CPW4), proc, _i32(0))
    plsc.subcore_barrier()
    for t in range(6):
        pltpu.sync_copy(accS.at[pl.ds(base_row + _i32(t * CH4), CH4)],
                        acc_hbm.at[cid, pl.ds(base_row + _i32(t * CH4), CH4)])
    pltpu.sync_copy(
        accS.at[pl.ds(base_row + _i32(6 * CH4), SPT - 6 * CH4)],
        acc_hbm.at[cid, pl.ds(base_row + _i32(6 * CH4), SPT - 6 * CH4)])


@functools.lru_cache(maxsize=1)
def _sc_kernels():
    mesh = plsc.VectorSubcoreMesh(core_axis_name="c", subcore_axis_name="s")
    k1 = pl.kernel(
        _k1_body,
        out_type=jax.ShapeDtypeStruct((TBL,), jnp.int32),
        mesh=mesh,
        scratch_types=[
            pltpu.VMEM((PW,), jnp.int32),
            pltpu.VMEM((PW,), jnp.int32),
            pltpu.VMEM((PW,), jnp.int32),
            pltpu.VMEM((PW,), jnp.int32),
            pltpu.SemaphoreType.DMA,
        ],
    )
    k2 = pl.kernel(
        _k2_body,
        out_type=jax.ShapeDtypeStruct((NC, NDEG), jnp.float32),
        mesh=mesh,
        scratch_types=[
            pltpu.VMEM((PW,), jnp.int32),
            pltpu.VMEM((PW,), jnp.int32),
            pltpu.VMEM((PW,), jnp.int32),
            pltpu.VMEM((PW,), jnp.float32),
            pltpu.VMEM((PW,), jnp.int32),
            pltpu.VMEM((PW,), jnp.int32),
            pltpu.VMEM((640,), jnp.float32),
            pltpu.VMEM_SHARED((NDEG,), jnp.float32),
            pltpu.SemaphoreType.DMA,
        ],
    )
    k4 = pl.kernel(
        _k4_body,
        out_type=jax.ShapeDtypeStruct((NC, NACC, C), jnp.float32),
        mesh=mesh,
        scratch_types=[
            pltpu.VMEM((CPW4, CH4), jnp.int32),
            pltpu.VMEM((2, CH4), jnp.int32),
            pltpu.VMEM((2, CH4, C), jnp.float32),
            pltpu.VMEM_SHARED((NACC, C), jnp.float32),
            pltpu.SemaphoreType.DMA,
            pltpu.SemaphoreType.DMA,
        ],
    )
    return k1, k2, k4


def _z():
    return jnp.int32(0)


_BLK = 1000


def _k3_body(degp_ref, x_ref, xs_ref, disb_ref):
    dp = degp_ref[...]
    deg = 1.0 + dp[:, 0:1] + dp[:, 1:2]
    dis = lax.rsqrt(deg)
    disb_ref[...] = jnp.broadcast_to(dis, (_BLK, C))
    xs_ref[...] = dis * x_ref[...]


def _k5_body(acc_ref, disb_ref, xs_ref, pan_ref, w1_ref, b1_ref,
             w2_ref, b2_ref, out_ref):
    a = acc_ref[...]
    s = a[0] + a[1]
    w0 = pan_ref[0, 0]
    w01 = pan_ref[0, 1]
    y = disb_ref[...] * (w01 * s + w0 * xs_ref[...])
    dn = (((1,), (1,)), ((), ()))
    z = lax.dot_general(y, w1_ref[...], dn,
                        preferred_element_type=jnp.float32) + b1_ref[...]
    h = jnp.where(z > 0, z, jnp.exp(z) - 1.0)
    out_ref[...] = lax.dot_general(h, w2_ref[...], dn,
                                   preferred_element_type=jnp.float32) + b2_ref[...]


_k3 = pl.pallas_call(
    _k3_body,
    grid=(N // _BLK,),
    in_specs=[
        pl.BlockSpec((_BLK, 2), lambda j: (j, _z())),
        pl.BlockSpec((_BLK, C), lambda j: (j, _z())),
    ],
    out_specs=[
        pl.BlockSpec((_BLK, C), lambda j: (j, _z())),
        pl.BlockSpec((_BLK, C), lambda j: (j, _z())),
    ],
    out_shape=[
        jax.ShapeDtypeStruct((N, C), jnp.float32),
        jax.ShapeDtypeStruct((N, C), jnp.float32),
    ],
)

_k5 = pl.pallas_call(
    _k5_body,
    grid=(N // _BLK,),
    in_specs=[
        pl.BlockSpec((NC, _BLK, C), lambda j: (_z(), j, _z())),
        pl.BlockSpec((_BLK, C), lambda j: (j, _z())),
        pl.BlockSpec((_BLK, C), lambda j: (j, _z())),
        pl.BlockSpec((1, 2), lambda j: (_z(), _z())),
        pl.BlockSpec((C, C), lambda j: (_z(), _z())),
        pl.BlockSpec((1, C), lambda j: (_z(), _z())),
        pl.BlockSpec((C, C), lambda j: (_z(), _z())),
        pl.BlockSpec((1, C), lambda j: (_z(), _z())),
    ],
    out_specs=pl.BlockSpec((_BLK, C), lambda j: (j, _z())),
    out_shape=jax.ShapeDtypeStruct((N, C), jnp.float32),
)


def kernel(x, edge_index, pan_weight, W1, b1, W2, b2):
    ei = edge_index.astype(jnp.int32)
    rows = jnp.concatenate(
        [ei[1], jnp.full((PAD,), N, jnp.int32)]).reshape(NW, PW)
    cols = jnp.concatenate(
        [ei[0], jnp.arange(PAD, dtype=jnp.int32)]).reshape(NW, PW)
    rows3 = rows.reshape(NW, CPW4, CH4)
    cols3 = cols.reshape(NW, CPW4, CH4)
    x = x.astype(jnp.float32)
    k1, k2, k4 = _sc_kernels()
    tbl = k1(rows, cols)
    degp = k2(rows, cols, tbl)
    xs, disb = _k3(degp.T, x)
    acc = k4(rows3, cols3, xs)
    pw = pan_weight.astype(jnp.float32)
    pan2 = jnp.stack([pw[0], pw[0] * pw[1]]).reshape(1, 2)
    out = _k5(acc, disb, xs, pan2,
              W1.astype(jnp.float32), b1.astype(jnp.float32).reshape(1, C),
              W2.astype(jnp.float32), b2.astype(jnp.float32).reshape(1, C))
    return out.astype(jnp.float64)


# speedup vs baseline: 190.4392x; 1.0045x over previous
import functools

import jax
import jax.numpy as jnp
from jax import lax
from jax.experimental import pallas as pl
from jax.experimental.pallas import tpu as pltpu
from jax.experimental.pallas import tpu_sc as plsc

N = 10000
E = 320000
C = 128
NC = 2
NS = 16
NW = NC * NS
CH = 96
CPW = 105
PW = CPW * CH
EP = NW * PW
PAD = EP - E
TBL = N * N + PAD
SPT = 632
NACC = NS * SPT
NDEG = 10240
NSTR = 7
SEG = PW // NSTR
CH4 = 96
CPW4 = PW // CH4


def _i32(v):
    return jnp.int32(v)


def _worker(cid, sid):
    return sid * _i32(NC) + cid


def _chunk_loop(body):
    def outer(j, _):
        def inner(k, _):
            body(j, k, pl.ds(k * _i32(16), 16))
            return _i32(0)
        return lax.fori_loop(_i32(0), _i32(CH // 16), inner, _i32(0))
    lax.fori_loop(_i32(0), _i32(CPW), outer, _i32(0))


def _compute_keys(rbuf, cbuf, kbuf):
    def body(j, k, sl):
        kbuf[j, sl] = rbuf[j, sl] * _i32(N) + cbuf[j, sl]
    _chunk_loop(body)
    return lax.iota(jnp.int32, 16)


def _k1_body(rows_hbm, cols_hbm, tbl_hbm, rbuf, cbuf, *rest):
    bufs, sem = rest[:2 * NSTR], rest[2 * NSTR]
    cid = lax.axis_index("c")
    sid = lax.axis_index("s")
    wid = _worker(cid, sid)
    pltpu.sync_copy(rows_hbm.at[wid], rbuf)
    pltpu.sync_copy(cols_hbm.at[wid], cbuf)
    iota = lax.iota(jnp.int32, 16)
    base = wid * _i32(PW)

    for q in range(NSTR):
        kq = bufs[2 * q]
        iq = bufs[2 * q + 1]
        off = _i32(q * SEG)

        def body(i, _, kq=kq, iq=iq, off=off):
            sl = pl.ds(i * _i32(16), 16)
            gsl = pl.ds(off + i * _i32(16), 16)
            kq[sl] = rbuf[gsl] * _i32(N) + cbuf[gsl]
            iq[sl] = base + off + i * _i32(16) + iota
            return _i32(0)

        lax.fori_loop(_i32(0), _i32(SEG // 16), body, _i32(0))
        pltpu.async_copy(iq, tbl_hbm.at[kq], sem)

    for q in range(NSTR):
        pltpu.make_async_copy(bufs[2 * q + 1],
                              tbl_hbm.at[bufs[2 * q]], sem).wait()


def _k2_body(rows_hbm, cols_hbm, tbl_hbm, degp_hbm,
             rbuf, cbuf, kbuf, vbuf, xbuf, gbuf, zbuf, degS, sem):
    cid = lax.axis_index("c")
    sid = lax.axis_index("s")
    wid = _worker(cid, sid)
    pltpu.sync_copy(rows_hbm.at[wid], rbuf)
    pltpu.sync_copy(cols_hbm.at[wid], cbuf)
    iota = lax.iota(jnp.int32, 16)
    base = wid * _i32(PW)

    def keys(i, _):
        sl = pl.ds(i * _i32(16), 16)
        kbuf[sl] = rbuf[sl] * _i32(N) + cbuf[sl]
        return _i32(0)

    lax.fori_loop(_i32(0), _i32(PW // 16), keys, _i32(0))
    gcopy = pltpu.async_copy(tbl_hbm.at[kbuf], gbuf, sem)

    def zero(i, _):
        zbuf[pl.ds(i * _i32(16), 16)] = jnp.zeros((16,), jnp.float32)
        return _i32(0)

    lax.fori_loop(_i32(0), _i32(40), zero, _i32(0))
    pltpu.sync_copy(zbuf, degS.at[pl.ds(sid * _i32(640), 640)])
    gcopy.wait()
    plsc.subcore_barrier()

    ones = jnp.full((16,), 1.0, jnp.float32)
    zeros = jnp.zeros((16,), jnp.float32)
    zi = jnp.zeros((16,), jnp.int32)

    def flags(i, _):
        sl = pl.ds(i * _i32(16), 16)
        r = rbuf[sl]
        c = cbuf[sl]
        g = gbuf[sl]
        myid = base + i * _i32(16) + iota
        valid = r < _i32(N)
        flag = (g == myid) & (r != c) & valid
        vbuf[sl] = jnp.where(flag, ones, zeros)
        xbuf[sl] = jnp.where(valid, r, zi)
        return _i32(0)

    lax.fori_loop(_i32(0), _i32(PW // 16), flags, _i32(0))
    pltpu.sync_copy(vbuf, degS.at[xbuf], add=True)
    plsc.subcore_barrier()
    pltpu.sync_copy(degS.at[pl.ds(sid * _i32(640), 640)],
                    degp_hbm.at[cid, pl.ds(sid * _i32(640), 640)])


def _k4_body(rows_hbm, cols_hbm, xs_hbm, acc_hbm,
             cbuf, ring, rowsbuf, accS, sem, sem2):
    cid = lax.axis_index("c")
    sid = lax.axis_index("s")
    wid = _worker(cid, sid)
    pltpu.sync_copy(cols_hbm.at[wid], cbuf)
    iota = lax.iota(jnp.int32, 16)

    def xform(b):
        def inner(k, _):
            sl = pl.ds(k * _i32(16), 16)
            r = ring[b, sl]
            ring[b, sl] = jnp.where(r < _i32(N), r, _i32(N) + iota)
            return _i32(0)
        lax.fori_loop(_i32(0), _i32(CH4 // 16), inner, _i32(0))

    z16 = jnp.zeros((16,), jnp.float32)

    def zero(i, _):
        def zin(k, _):
            rowsbuf[_i32(0), i, pl.ds(k * _i32(16), 16)] = z16
            return _i32(0)
        return lax.fori_loop(_i32(0), _i32(8), zin, _i32(0))

    lax.fori_loop(_i32(0), _i32(CH4), zero, _i32(0))
    base_row = sid * _i32(SPT)
    for t in range(6):
        pltpu.sync_copy(rowsbuf.at[_i32(0)],
                        accS.at[pl.ds(base_row + _i32(t * CH4), CH4)])
    pltpu.sync_copy(rowsbuf.at[_i32(0), pl.ds(0, SPT - 6 * CH4)],
                    accS.at[pl.ds(base_row + _i32(6 * CH4), SPT - 6 * CH4)])
    plsc.subcore_barrier()

    pltpu.sync_copy(rows_hbm.at[wid, _i32(0)], ring.at[_i32(0)])
    xform(_i32(0))
    pltpu.async_copy(xs_hbm.at[cbuf.at[_i32(0)]], rowsbuf.at[_i32(0)], sem)

    def proc(j, _):
        b = j & _i32(1)
        nxt = j + _i32(1)
        bn = nxt & _i32(1)

        @pl.when(nxt < _i32(CPW4))
        def _():
            pltpu.async_copy(xs_hbm.at[cbuf.at[nxt]], rowsbuf.at[bn], sem)
            pltpu.async_copy(rows_hbm.at[wid, nxt], ring.at[bn], sem2)

        pltpu.make_async_copy(xs_hbm.at[cbuf.at[j]],
                              rowsbuf.at[b], sem).wait()
        pltpu.sync_copy(rowsbuf.at[b], accS.at[ring.at[b]], add=True)

        @pl.when(nxt < _i32(CPW4))
        def _():
            pltpu.make_async_copy(rows_hbm.at[wid, nxt],
                                  ring.at[bn], sem2).wait()
            xform(bn)

        return _i32(0)

    lax.fori_loop(_i32(0), _i32(CPW4), proc, _i32(0))
    plsc.subcore_barrier()
    for t in range(6):
        pltpu.sync_copy(accS.at[pl.ds(base_row + _i32(t * CH4), CH4)],
                        acc_hbm.at[cid, pl.ds(base_row + _i32(t * CH4), CH4)])
    pltpu.sync_copy(
        accS.at[pl.ds(base_row + _i32(6 * CH4), SPT - 6 * CH4)],
        acc_hbm.at[cid, pl.ds(base_row + _i32(6 * CH4), SPT - 6 * CH4)])


@functools.lru_cache(maxsize=1)
def _sc_kernels():
    mesh = plsc.VectorSubcoreMesh(core_axis_name="c", subcore_axis_name="s")
    k1 = pl.kernel(
        _k1_body,
        out_type=jax.ShapeDtypeStruct((TBL,), jnp.int32),
        mesh=mesh,
        scratch_types=[pltpu.VMEM((PW,), jnp.int32),
                       pltpu.VMEM((PW,), jnp.int32)]
                      + [pltpu.VMEM((SEG,), jnp.int32)] * (2 * NSTR)
                      + [pltpu.SemaphoreType.DMA],
    )
    k2 = pl.kernel(
        _k2_body,
        out_type=jax.ShapeDtypeStruct((NC, NDEG), jnp.float32),
        mesh=mesh,
        scratch_types=[
            pltpu.VMEM((PW,), jnp.int32),
            pltpu.VMEM((PW,), jnp.int32),
            pltpu.VMEM((PW,), jnp.int32),
            pltpu.VMEM((PW,), jnp.float32),
            pltpu.VMEM((PW,), jnp.int32),
            pltpu.VMEM((PW,), jnp.int32),
            pltpu.VMEM((640,), jnp.float32),
            pltpu.VMEM_SHARED((NDEG,), jnp.float32),
            pltpu.SemaphoreType.DMA,
        ],
    )
    k4 = pl.kernel(
        _k4_body,
        out_type=jax.ShapeDtypeStruct((NC, NACC, C), jnp.float32),
        mesh=mesh,
        scratch_types=[
            pltpu.VMEM((CPW4, CH4), jnp.int32),
            pltpu.VMEM((2, CH4), jnp.int32),
            pltpu.VMEM((2, CH4, C), jnp.float32),
            pltpu.VMEM_SHARED((NACC, C), jnp.float32),
            pltpu.SemaphoreType.DMA,
            pltpu.SemaphoreType.DMA,
        ],
    )
    return k1, k2, k4


def _z():
    return jnp.int32(0)


_BLK = 1000


def _k3_body(degp_ref, x_ref, xs_ref, disb_ref):
    dp = degp_ref[...]
    deg = 1.0 + dp[:, 0:1] + dp[:, 1:2]
    dis = lax.rsqrt(deg)
    disb_ref[...] = jnp.broadcast_to(dis, (_BLK, C))
    xs_ref[...] = dis * x_ref[...]


def _k5_body(acc_ref, disb_ref, xs_ref, pan_ref, w1_ref, b1_ref,
             w2_ref, b2_ref, out_ref):
    a = acc_ref[...]
    s = a[0] + a[1]
    w0 = pan_ref[0, 0]
    w01 = pan_ref[0, 1]
    y = disb_ref[...] * (w01 * s + w0 * xs_ref[...])
    dn = (((1,), (1,)), ((), ()))
    z = lax.dot_general(y, w1_ref[...], dn,
                        preferred_element_type=jnp.float32) + b1_ref[...]
    h = jnp.where(z > 0, z, jnp.exp(z) - 1.0)
    out_ref[...] = lax.dot_general(h, w2_ref[...], dn,
                                   preferred_element_type=jnp.float32) + b2_ref[...]


_k3 = pl.pallas_call(
    _k3_body,
    grid=(N // _BLK,),
    in_specs=[
        pl.BlockSpec((_BLK, 2), lambda j: (j, _z())),
        pl.BlockSpec((_BLK, C), lambda j: (j, _z())),
    ],
    out_specs=[
        pl.BlockSpec((_BLK, C), lambda j: (j, _z())),
        pl.BlockSpec((_BLK, C), lambda j: (j, _z())),
    ],
    out_shape=[
        jax.ShapeDtypeStruct((N, C), jnp.float32),
        jax.ShapeDtypeStruct((N, C), jnp.float32),
    ],
)

_k5 = pl.pallas_call(
    _k5_body,
    grid=(N // _BLK,),
    in_specs=[
        pl.BlockSpec((NC, _BLK, C), lambda j: (_z(), j, _z())),
        pl.BlockSpec((_BLK, C), lambda j: (j, _z())),
        pl.BlockSpec((_BLK, C), lambda j: (j, _z())),
        pl.BlockSpec((1, 2), lambda j: (_z(), _z())),
        pl.BlockSpec((C, C), lambda j: (_z(), _z())),
        pl.BlockSpec((1, C), lambda j: (_z(), _z())),
        pl.BlockSpec((C, C), lambda j: (_z(), _z())),
        pl.BlockSpec((1, C), lambda j: (_z(), _z())),
    ],
    out_specs=pl.BlockSpec((_BLK, C), lambda j: (j, _z())),
    out_shape=jax.ShapeDtypeStruct((N, C), jnp.float32),
)


def kernel(x, edge_index, pan_weight, W1, b1, W2, b2):
    ei = edge_index.astype(jnp.int32)
    rows = jnp.concatenate(
        [ei[1], jnp.full((PAD,), N, jnp.int32)]).reshape(NW, PW)
    cols = jnp.concatenate(
        [ei[0], jnp.arange(PAD, dtype=jnp.int32)]).reshape(NW, PW)
    rows3 = rows.reshape(NW, CPW4, CH4)
    cols3 = cols.reshape(NW, CPW4, CH4)
    x = x.astype(jnp.float32)
    k1, k2, k4 = _sc_kernels()
    tbl = k1(rows, cols)
    degp = k2(rows, cols, tbl)
    xs, disb = _k3(degp.T, x)
    acc = k4(rows3, cols3, xs)
    pw = pan_weight.astype(jnp.float32)
    pan2 = jnp.stack([pw[0], pw[0] * pw[1]]).reshape(1, 2)
    out = _k5(acc, disb, xs, pan2,
              W1.astype(jnp.float32), b1.astype(jnp.float32).reshape(1, C),
              W2.astype(jnp.float32), b2.astype(jnp.float32).reshape(1, C))
    return out.astype(jnp.float64)
